# baseline (speedup 1.0000x reference)
import functools

import jax
import jax.numpy as jnp
import numpy as np
from jax import lax
from jax.experimental import pallas as pl
from jax.experimental.pallas import tpu as pltpu
from jax.experimental.pallas import tpu_sc as plsc

NC = 2
NS = 16
NW = NC * NS
L = 16
C = 128
G = 3
S = 1


def _vreg_gather(vec, idx):
    dnums = lax.GatherDimensionNumbers(
        offset_dims=(), collapsed_slice_dims=(0,), start_index_map=(0,))
    return lax.gather(vec, idx[:, None], dnums, slice_sizes=(1,),
                      mode=lax.GatherScatterMode.PROMISE_IN_BOUNDS)


def _bf16_bits(u):
    return (u + 0x7FFF + ((u >> 16) & 1)) >> 16


def _transpose_body(x_ref, o_ref):
    y = x_ref[...].T
    h = y.shape[1] // 4
    lo = jnp.concatenate([y[:, 0:h], y[:, 2 * h:3 * h]], axis=1)
    hi = jnp.concatenate([y[:, h:2 * h], y[:, 3 * h:4 * h]], axis=1)
    ulo = _bf16_bits(lax.bitcast_convert_type(lo, jnp.uint32))
    uhi = _bf16_bits(lax.bitcast_convert_type(hi, jnp.uint32))
    o_ref[...] = lax.bitcast_convert_type(ulo | (uhi << 16), jnp.int32)


def _combine_body(p_ref, b_ref, o_ref):
    s = p_ref[0] + p_ref[1]
    o_ref[...] = s.T + b_ref[...][None, :]


def _scale_unpacked(gbufs, sbufs, valv, b, s):
    def group(g, _):
        vals16 = valv[pl.ds(g * L, L)]
        for e in range(L):
            bv = _vreg_gather(vals16, jnp.full((L,), e, jnp.int32))
            row = g * L + e
            for j in range(2):
                xi = gbufs[b, row, pl.ds(j * L, L)]
                lo = plsc.bitcast(lax.shift_left(xi, 16), jnp.float32)
                hi = plsc.bitcast(
                    lax.bitwise_and(xi, jnp.int32(-65536)), jnp.float32)
                sbufs[s, row, pl.ds(j * 2 * L, L)] = lo * bv
                sbufs[s, row, pl.ds(j * 2 * L + L, L)] = hi * bv
        return _
    lax.fori_loop(0, C // L, group, None)


def _sc_spmm_body(nchunks, xt_hbm, val_hbm, row_hbm, col_hbm, out_hbm,
                  acc, table, colv, rowv, valv, gbufs, sbufs, sload, sg, ss):
    n = acc.shape[0]
    rps = n // NS
    cid = lax.axis_index("c")
    sid = lax.axis_index("s")
    wid = cid * NS + sid

    dc = pltpu.async_copy(col_hbm.at[wid], colv, sload)
    dt = pltpu.async_copy(xt_hbm.at[pl.ds(sid * rps, rps)],
                          table.at[pl.ds(sid * rps, rps)], sload)

    def gather(t, b):
        return pltpu.async_copy(table.at[colv.at[t]], gbufs.at[b], sg[b])

    dc.wait()

    def zero_row(i, _):
        for j in range(4):
            sbufs[0, i, pl.ds(L * j, L)] = jnp.zeros((L,), jnp.float32)
        return _
    lax.fori_loop(0, C, zero_row, None)
    for k in range(rps // C):
        pltpu.sync_copy(sbufs.at[0], acc.at[pl.ds(sid * rps + k * C, C)])
    dt.wait()
    plsc.subcore_barrier()
    for b in range(G):
        gather(b, b)

    def scatter(t, s):
        return pltpu.async_copy(sbufs.at[s], acc.at[rowv], ss[s],
                                add=True)

    def scatter_wait(s):
        pltpu.make_async_copy(sbufs.at[s], acc.at[rowv], ss[s]).wait()

    nq = nchunks // G

    def round_(q, _):
        for b in range(G):
            t = q * G + b
            s = 0
            pltpu.make_async_copy(table.at[colv.at[t]], gbufs.at[b],
                                  sg[b]).wait()
            if b == 0:
                @pl.when(q > 0)
                def _():
                    scatter_wait(s)
            else:
                scatter_wait(s)
            pltpu.sync_copy(val_hbm.at[wid, t], valv)
            pltpu.sync_copy(row_hbm.at[wid, t], rowv)
            _scale_unpacked(gbufs, sbufs, valv, b, s)
            scatter(t, s)

            @pl.when(q < nq - 1)
            def _():
                gather(q * G + b + G, b)
        return _
    lax.fori_loop(0, nq, round_, None)

    for s in range(S):
        scatter_wait(s)
    plsc.subcore_barrier()
    pltpu.sync_copy(acc.at[pl.ds(sid * rps, rps)],
                    out_hbm.at[cid, pl.ds(sid * rps, rps)])


def kernel(x, values, biases, rows, cols):
    b, n = x.shape
    nnz = values.shape[0]

    nchunks = -(-nnz // (NW * C * G)) * G
    nnz_pad = nchunks * C * NW
    pad = nnz_pad - nnz
    valp = jnp.concatenate([values, jnp.zeros((pad,), values.dtype)])
    rowp = jnp.concatenate([rows, jnp.zeros((pad,), rows.dtype)])
    colp = jnp.concatenate([cols, jnp.zeros((pad,), cols.dtype)])
    valp = valp.reshape(NW, nchunks, C)
    rowp = rowp.reshape(NW, nchunks, C)
    colp = colp.reshape(NW, nchunks, C)

    blk = 512
    xt = pl.pallas_call(
        _transpose_body,
        grid=(n // blk,),
        in_specs=[pl.BlockSpec((b, blk), lambda i: (0, i))],
        out_specs=pl.BlockSpec((blk, b // 2), lambda i: (i, 0)),
        out_shape=jax.ShapeDtypeStruct((n, b // 2), jnp.int32),
    )(x)

    sc_spmm = pl.kernel(
        functools.partial(_sc_spmm_body, nchunks),
        out_type=jax.ShapeDtypeStruct((NC, n, b), jnp.float32),
        mesh=plsc.VectorSubcoreMesh(core_axis_name="c",
                                    subcore_axis_name="s"),
        compiler_params=pltpu.CompilerParams(use_tc_tiling_on_sc=False,
                                             needs_layout_passes=False),
        scratch_types=[
            pltpu.VMEM_SHARED((n, b), jnp.float32),
            pltpu.VMEM_SHARED((n, b // 2), jnp.int32),
            pltpu.VMEM((nchunks, C), jnp.int32),
            pltpu.VMEM((C,), jnp.int32),
            pltpu.VMEM((C,), jnp.float32),
            pltpu.VMEM((G, C, b // 2), jnp.int32),
            pltpu.VMEM((S, C, b), jnp.float32),
            pltpu.SemaphoreType.DMA,
            [pltpu.SemaphoreType.DMA] * G,
            [pltpu.SemaphoreType.DMA] * S,
        ],
    )
    partials = sc_spmm(xt, valp, rowp, colp)

    out = pl.pallas_call(
        _combine_body,
        grid=(n // blk,),
        in_specs=[
            pl.BlockSpec((NC, blk, b), lambda i: (0, i, 0)),
            pl.BlockSpec((blk,), lambda i: (i,)),
        ],
        out_specs=pl.BlockSpec((b, blk), lambda i: (0, i)),
        out_shape=jax.ShapeDtypeStruct((b, n), jnp.float32),
    )(partials, biases)
    return out


# speedup vs baseline: 1.3719x; 1.3719x over previous
import functools

import jax
import jax.numpy as jnp
import numpy as np
from jax import lax
from jax.experimental import pallas as pl
from jax.experimental.pallas import tpu as pltpu
from jax.experimental.pallas import tpu_sc as plsc

NC = 2
NS = 16
NW = NC * NS
L = 16
C = 96
G = 6
S = 2


def _vreg_gather(vec, idx):
    dnums = lax.GatherDimensionNumbers(
        offset_dims=(), collapsed_slice_dims=(0,), start_index_map=(0,))
    return lax.gather(vec, idx[:, None], dnums, slice_sizes=(1,),
                      mode=lax.GatherScatterMode.PROMISE_IN_BOUNDS)


def _bf16_bits(u):
    return (u + 0x7FFF + ((u >> 16) & 1)) >> 16


def _transpose_body(x_ref, o_ref):
    y = x_ref[...].T
    h = y.shape[1] // 4
    lo = jnp.concatenate([y[:, 0:h], y[:, 2 * h:3 * h]], axis=1)
    hi = jnp.concatenate([y[:, h:2 * h], y[:, 3 * h:4 * h]], axis=1)
    ulo = _bf16_bits(lax.bitcast_convert_type(lo, jnp.uint32))
    uhi = _bf16_bits(lax.bitcast_convert_type(hi, jnp.uint32))
    o_ref[...] = lax.bitcast_convert_type(ulo | (uhi << 16), jnp.int32)


def _combine_body(p_ref, b_ref, o_ref):
    s = p_ref[0] + p_ref[1]
    o_ref[...] = s.T + b_ref[...][None, :]


def _scale_unpacked(gbufs, sbufs, valv, t, b, s):
    def group(g, _):
        vals16 = valv[t, pl.ds(g * L, L)]
        for e in range(L):
            bv = _vreg_gather(vals16, jnp.full((L,), e, jnp.int32))
            row = g * L + e
            for j in range(2):
                xi = gbufs[b, row, pl.ds(j * L, L)]
                lo = plsc.bitcast(lax.shift_left(xi, 16), jnp.float32)
                hi = plsc.bitcast(
                    lax.bitwise_and(xi, jnp.int32(-65536)), jnp.float32)
                sbufs[s, row, pl.ds(j * 2 * L, L)] = lo * bv
                sbufs[s, row, pl.ds(j * 2 * L + L, L)] = hi * bv
        return _
    lax.fori_loop(0, C // L, group, None)


def _sc_spmm_body(nchunks, xt_hbm, val_hbm, row_hbm, col_hbm, out_hbm,
                  acc, colv, rowv, valv, gbufs, sbufs, sload, sg, ss):
    n = acc.shape[0]
    rps = n // NS
    cid = lax.axis_index("c")
    sid = lax.axis_index("s")
    wid = cid * NS + sid

    dc = pltpu.async_copy(col_hbm.at[wid], colv, sload)
    dr = pltpu.async_copy(row_hbm.at[wid], rowv, sload)
    dv = pltpu.async_copy(val_hbm.at[wid], valv, sload)

    def gather(t, b):
        return pltpu.async_copy(xt_hbm.at[colv.at[t]], gbufs.at[b], sg[b])

    dc.wait()
    for b in range(G):
        gather(b, b)

    def zero_row(i, _):
        for j in range(4):
            sbufs[0, i, pl.ds(L * j, L)] = jnp.zeros((L,), jnp.float32)
        return _
    lax.fori_loop(0, C, zero_row, None)
    for k in range(rps // 64):
        pltpu.sync_copy(sbufs.at[0, pl.ds(0, 64)],
                        acc.at[pl.ds(sid * rps + k * 64, 64)])
    dr.wait(); dv.wait()
    plsc.subcore_barrier()

    def scatter(t, s):
        return pltpu.async_copy(sbufs.at[s], acc.at[rowv.at[t]], ss[s],
                                add=True)

    def scatter_wait(s):
        pltpu.make_async_copy(sbufs.at[s], acc.at[rowv.at[0]],
                              ss[s]).wait()

    nq = nchunks // G

    def round_(q, _):
        for b in range(G):
            t = q * G + b
            s = b % S
            pltpu.make_async_copy(xt_hbm.at[colv.at[t]], gbufs.at[b],
                                  sg[b]).wait()
            if b < S:
                @pl.when(q > 0)
                def _():
                    scatter_wait(s)
            else:
                scatter_wait(s)
            _scale_unpacked(gbufs, sbufs, valv, t, b, s)
            scatter(t, s)

            @pl.when(q < nq - 1)
            def _():
                gather(q * G + b + G, b)
        return _
    lax.fori_loop(0, nq, round_, None)

    for s in range(S):
        scatter_wait(s)
    plsc.subcore_barrier()
    pltpu.sync_copy(acc.at[pl.ds(sid * rps, rps)],
                    out_hbm.at[cid, pl.ds(sid * rps, rps)])


def kernel(x, values, biases, rows, cols):
    b, n = x.shape
    nnz = values.shape[0]

    nchunks = -(-nnz // (NW * C * G)) * G
    nnz_pad = nchunks * C * NW
    pad = nnz_pad - nnz
    valp = jnp.concatenate([values, jnp.zeros((pad,), values.dtype)])
    rowp = jnp.concatenate([rows, jnp.zeros((pad,), rows.dtype)])
    colp = jnp.concatenate([cols, jnp.zeros((pad,), cols.dtype)])
    valp = valp.reshape(NW, nchunks, C)
    rowp = rowp.reshape(NW, nchunks, C)
    colp = colp.reshape(NW, nchunks, C)

    blk = 512
    xt = pl.pallas_call(
        _transpose_body,
        grid=(n // blk,),
        in_specs=[pl.BlockSpec((b, blk), lambda i: (0, i))],
        out_specs=pl.BlockSpec((blk, b // 2), lambda i: (i, 0)),
        out_shape=jax.ShapeDtypeStruct((n, b // 2), jnp.int32),
    )(x)

    sc_spmm = pl.kernel(
        functools.partial(_sc_spmm_body, nchunks),
        out_type=jax.ShapeDtypeStruct((NC, n, b), jnp.float32),
        mesh=plsc.VectorSubcoreMesh(core_axis_name="c",
                                    subcore_axis_name="s"),
        compiler_params=pltpu.CompilerParams(use_tc_tiling_on_sc=False,
                                             needs_layout_passes=False),
        scratch_types=[
            pltpu.VMEM_SHARED((n, b), jnp.float32),
            pltpu.VMEM((nchunks, C), jnp.int32),
            pltpu.VMEM((nchunks, C), jnp.int32),
            pltpu.VMEM((nchunks, C), jnp.float32),
            pltpu.VMEM((G, C, b // 2), jnp.int32),
            pltpu.VMEM((S, C, b), jnp.float32),
            pltpu.SemaphoreType.DMA,
            [pltpu.SemaphoreType.DMA] * G,
            [pltpu.SemaphoreType.DMA] * S,
        ],
    )
    partials = sc_spmm(xt, valp, rowp, colp)

    out = pl.pallas_call(
        _combine_body,
        grid=(n // blk,),
        in_specs=[
            pl.BlockSpec((NC, blk, b), lambda i: (0, i, 0)),
            pl.BlockSpec((blk,), lambda i: (i,)),
        ],
        out_specs=pl.BlockSpec((b, blk), lambda i: (0, i)),
        out_shape=jax.ShapeDtypeStruct((b, n), jnp.float32),
    )(partials, biases)
    return out
